# Initial kernel scaffold; baseline (speedup 1.0000x reference)
import functools

import jax
import jax.numpy as jnp
import numpy as np
from jax import lax
from jax.experimental import pallas as pl
from jax.experimental.pallas import tpu as pltpu
from jax.experimental.pallas import tpu_sc as plsc

NC = 2
NS = 16
LN = 16

_BNR = float(1.0 / np.sqrt(1.0 + 1e-5))


def _edge_chunk(e_per_tile):
    for k in range(128, 0, -1):
        if e_per_tile % k == 0:
            return k
    return 1


def _zero_rows(zb_ref, rows, width):
    z16 = jnp.zeros((LN,), jnp.float32)

    def body(j, carry):
        for k in range(width // LN):
            zb_ref[j, pl.ds(k * LN, LN)] = z16
        return carry

    lax.fori_loop(0, rows, body, 0)


def _sc_degree(d1s, d2s, n):
    ns, npc, k = d1s.shape
    stripe = n // NS
    ones_hbm = jnp.ones((k, LN), jnp.float32)
    mesh = plsc.VectorSubcoreMesh(core_axis_name="c", subcore_axis_name="s",
                                  num_cores=NC, num_subcores=NS)

    @functools.partial(
        pl.kernel,
        out_type=jax.ShapeDtypeStruct((NC, n, LN), jnp.float32),
        mesh=mesh,
        scratch_types=[
            pltpu.VMEM((npc, k), jnp.int32),
            pltpu.VMEM((k, LN), jnp.float32),
            pltpu.VMEM((stripe, LN), jnp.float32),
            pltpu.VMEM_SHARED((n, LN), jnp.float32),
        ],
    )
    def deg_kernel(d1_hbm, d2_hbm, ones_h, out_hbm, slab_v, ones_v, zb_v, acc_sh):
        c = lax.axis_index("c")
        s = lax.axis_index("s")
        _zero_rows(zb_v, stripe, LN)
        pltpu.sync_copy(ones_h, ones_v)

        @pl.when(c == 0)
        def _():
            pltpu.sync_copy(d1_hbm.at[s], slab_v)

        @pl.when(c == 1)
        def _():
            pltpu.sync_copy(d2_hbm.at[s], slab_v)

        pltpu.sync_copy(zb_v, acc_sh.at[pl.ds(s * stripe, stripe)])
        plsc.subcore_barrier()

        def body(j, carry):
            pltpu.sync_copy(ones_v, acc_sh.at[slab_v.at[j]], add=True)
            return carry

        lax.fori_loop(0, npc, body, 0)
        plsc.subcore_barrier()
        pltpu.sync_copy(acc_sh.at[pl.ds(s * stripe, stripe)],
                        out_hbm.at[c, pl.ds(s * stripe, stripe)])

    return deg_kernel(d1s, d2s, ones_hbm)


def _sc_aggregate(mta, mtb, srcs, dsts):
    n, hh = mta.shape
    ns, npc, k = srcs.shape
    stripe = n // NS
    zr = _edge_chunk(stripe)
    mesh = plsc.VectorSubcoreMesh(core_axis_name="c", subcore_axis_name="s",
                                  num_cores=NC, num_subcores=NS)

    @functools.partial(
        pl.kernel,
        out_type=[jax.ShapeDtypeStruct((n, hh), jnp.float32),
                  jax.ShapeDtypeStruct((n, hh), jnp.float32)],
        mesh=mesh,
        scratch_types=[
            pltpu.VMEM((npc, k), jnp.int32),
            pltpu.VMEM((npc, k), jnp.int32),
            pltpu.VMEM((k, hh), jnp.float32),
            pltpu.VMEM((zr, hh), jnp.float32),
            pltpu.VMEM_SHARED((n, hh), jnp.float32),
            pltpu.SemaphoreType.DMA,
        ],
    )
    def agg_kernel(mta_hbm, mtb_hbm, src_hbm, dst_hbm, outa_hbm, outb_hbm,
                   srcs_v, dsts_v, rows_v, zb_v, acc_sh, sem):
        c = lax.axis_index("c")
        s = lax.axis_index("s")
        _zero_rows(zb_v, zr, hh)
        pltpu.sync_copy(src_hbm.at[s], srcs_v)
        pltpu.sync_copy(dst_hbm.at[s], dsts_v)
        for t in range(stripe // zr):
            pltpu.sync_copy(zb_v, acc_sh.at[pl.ds(s * stripe + t * zr, zr)])
        plsc.subcore_barrier()

        def make_body(mt_hbm):
            def body(j, carry):
                pltpu.async_copy(mt_hbm.at[srcs_v.at[j]], rows_v, sem).wait()
                pltpu.sync_copy(rows_v, acc_sh.at[dsts_v.at[j]], add=True)
                return carry
            return body

        @pl.when(c == 0)
        def _():
            lax.fori_loop(0, npc, make_body(mta_hbm), 0)

        @pl.when(c == 1)
        def _():
            lax.fori_loop(0, npc, make_body(mtb_hbm), 0)

        plsc.subcore_barrier()

        @pl.when(c == 0)
        def _():
            pltpu.sync_copy(acc_sh.at[pl.ds(s * stripe, stripe)],
                            outa_hbm.at[pl.ds(s * stripe, stripe)])

        @pl.when(c == 1)
        def _():
            pltpu.sync_copy(acc_sh.at[pl.ds(s * stripe, stripe)],
                            outb_hbm.at[pl.ds(s * stripe, stripe)])

    return agg_kernel(mta, mtb, srcs, dsts)


_R = 2000


def _row_spec(r, width):
    return pl.BlockSpec((r, width), lambda i: (i, 0))


def _full_spec(shape):
    nd = len(shape)
    return pl.BlockSpec(shape, lambda i: (0,) * nd)


def _tc_dinv(deg16):
    nc, n, w = deg16.shape
    r = min(_R, n)

    def body(deg_ref, o_ref):
        o_ref[...] = lax.rsqrt(deg_ref[:, :, 0:1] + 1.0)

    return pl.pallas_call(
        body,
        grid=(nc, n // r),
        in_specs=[pl.BlockSpec((1, r, w), lambda c, i: (c, i, 0))],
        out_specs=pl.BlockSpec((1, r, 1), lambda c, i: (c, i, 0)),
        out_shape=jax.ShapeDtypeStruct((nc, n, 1), jnp.float32),
    )(deg16)


def _tc_input(x, w_in, b_in):
    n, d = x.shape
    h = w_in.shape[1]
    hh = h // 2
    r = min(_R, n)

    def body(x_ref, w_ref, b_ref, oa_ref, ob_ref):
        hm = jnp.dot(x_ref[...], w_ref[...],
                     preferred_element_type=jnp.float32) + b_ref[...]
        oa_ref[...] = hm[:, :hh]
        ob_ref[...] = hm[:, hh:]

    return pl.pallas_call(
        body,
        grid=(n // r,),
        in_specs=[_row_spec(r, d), _full_spec((d, h)), _full_spec((1, h))],
        out_specs=[_row_spec(r, hh), _row_spec(r, hh)],
        out_shape=[jax.ShapeDtypeStruct((n, hh), jnp.float32),
                   jax.ShapeDtypeStruct((n, hh), jnp.float32)],
    )(x, w_in, b_in)


def _tc_mt0(h0a, h0b, w, dinv):
    n, hh = h0a.shape
    h = w.shape[0]
    r = min(_R, n)

    def body(ha_ref, hb_ref, w_ref, dv_ref, oa_ref, ob_ref):
        m = (jnp.dot(ha_ref[...], w_ref[:hh, :],
                     preferred_element_type=jnp.float32) +
             jnp.dot(hb_ref[...], w_ref[hh:, :],
                     preferred_element_type=jnp.float32))
        mt = dv_ref[...] * m
        oa_ref[...] = mt[:, :hh]
        ob_ref[...] = mt[:, hh:]

    return pl.pallas_call(
        body,
        grid=(n // r,),
        in_specs=[_row_spec(r, hh), _row_spec(r, hh), _full_spec((h, h)),
                  _row_spec(r, 1)],
        out_specs=[_row_spec(r, hh), _row_spec(r, hh)],
        out_shape=[jax.ShapeDtypeStruct((n, hh), jnp.float32),
                   jax.ShapeDtypeStruct((n, hh), jnp.float32)],
    )(h0a, h0b, w, dinv)


def _tc_layer(agga, aggb, mta, mtb, dv_p, dv_n, b, g, bet, w):
    n, hh = agga.shape
    h = w.shape[0]
    r = min(_R, n)

    def body(aa_ref, ab_ref, ma_ref, mb_ref, dp_ref, dn_ref,
             b_ref, g_ref, t_ref, w_ref, oa_ref, ob_ref):
        sc = g_ref[...] * _BNR
        dp = dp_ref[...]
        ha = (dp * (aa_ref[...] + ma_ref[...]) + b_ref[:, :hh]) * sc[:, :hh] \
            + t_ref[:, :hh]
        hb = (dp * (ab_ref[...] + mb_ref[...]) + b_ref[:, hh:]) * sc[:, hh:] \
            + t_ref[:, hh:]
        ha = jnp.maximum(ha, 0.0)
        hb = jnp.maximum(hb, 0.0)
        m = (jnp.dot(ha, w_ref[:hh, :], preferred_element_type=jnp.float32) +
             jnp.dot(hb, w_ref[hh:, :], preferred_element_type=jnp.float32))
        mt = dn_ref[...] * m
        oa_ref[...] = mt[:, :hh]
        ob_ref[...] = mt[:, hh:]

    return pl.pallas_call(
        body,
        grid=(n // r,),
        in_specs=[_row_spec(r, hh), _row_spec(r, hh),
                  _row_spec(r, hh), _row_spec(r, hh),
                  _row_spec(r, 1), _row_spec(r, 1),
                  _full_spec((1, h)), _full_spec((1, h)), _full_spec((1, h)),
                  _full_spec((h, h))],
        out_specs=[_row_spec(r, hh), _row_spec(r, hh)],
        out_shape=[jax.ShapeDtypeStruct((n, hh), jnp.float32),
                   jax.ShapeDtypeStruct((n, hh), jnp.float32)],
    )(agga, aggb, mta, mtb, dv_p, dv_n, b, g, bet, w)


def _tc_final(agga, aggb, mta, mtb, dv_p, b, g, bet, w_out_p, b_out_p, c_out):
    n, hh = agga.shape
    h = w_out_p.shape[0]
    cp = w_out_p.shape[1]
    r = min(_R, n)

    def body(aa_ref, ab_ref, ma_ref, mb_ref, dp_ref,
             b_ref, g_ref, t_ref, wo_ref, bo_ref, o_ref):
        sc = g_ref[...] * _BNR
        dp = dp_ref[...]
        ha = (dp * (aa_ref[...] + ma_ref[...]) + b_ref[:, :hh]) * sc[:, :hh] \
            + t_ref[:, :hh]
        hb = (dp * (ab_ref[...] + mb_ref[...]) + b_ref[:, hh:]) * sc[:, hh:] \
            + t_ref[:, hh:]
        ha = jnp.maximum(ha, 0.0)
        hb = jnp.maximum(hb, 0.0)
        logits = (jnp.dot(ha, wo_ref[:hh, :],
                          preferred_element_type=jnp.float32) +
                  jnp.dot(hb, wo_ref[hh:, :],
                          preferred_element_type=jnp.float32) + bo_ref[...])
        mx = jnp.max(logits, axis=1, keepdims=True)
        sh = logits - mx
        lse = jnp.log(jnp.sum(jnp.exp(sh), axis=1, keepdims=True))
        o_ref[...] = (sh - lse)[:, :c_out]

    return pl.pallas_call(
        body,
        grid=(n // r,),
        in_specs=[_row_spec(r, hh), _row_spec(r, hh),
                  _row_spec(r, hh), _row_spec(r, hh),
                  _row_spec(r, 1),
                  _full_spec((1, h)), _full_spec((1, h)), _full_spec((1, h)),
                  _full_spec((h, cp)), _full_spec((1, cp))],
        out_specs=_row_spec(r, c_out),
        out_shape=jax.ShapeDtypeStruct((n, c_out), jnp.float32),
    )(agga, aggb, mta, mtb, dv_p, b, g, bet, w_out_p, b_out_p)


def kernel(x, sample1_adj, sample2_adj, W_in, b_in, W_convs, b_convs,
           gammas, betas, W_out, b_out):
    n, d = x.shape
    h = W_in.shape[1]
    nlayers = W_convs.shape[0]
    c_out = W_out.shape[1]
    e = sample1_adj.shape[1]
    ept = e // NS
    k = _edge_chunk(ept)
    npc = ept // k

    def slab(a):
        return a.reshape(NS, npc, k)

    s1, d1 = slab(sample1_adj[0]), slab(sample1_adj[1])
    s2, d2 = slab(sample2_adj[0]), slab(sample2_adj[1])

    deg16 = _sc_degree(d1, d2, n)
    dinv_all = _tc_dinv(deg16)
    dinv1 = dinv_all[0]
    dinv2 = dinv_all[1]

    h0a, h0b = _tc_input(x, W_in, b_in.reshape(1, h))
    mta, mtb = _tc_mt0(h0a, h0b, W_convs[0], dinv1)

    cp = 128
    w_out_p = jnp.zeros((h, cp), jnp.float32).at[:, :c_out].set(W_out)
    b_out_p = jnp.full((1, cp), -1e30, jnp.float32).at[0, :c_out].set(b_out)

    half = nlayers // 2
    out = None
    for i in range(nlayers):
        srcs, dsts = (s1, d1) if i < half else (s2, d2)
        dv_p = dinv1 if i < half else dinv2
        agga, aggb = _sc_aggregate(mta, mtb, srcs, dsts)
        bi = b_convs[i].reshape(1, h)
        gi = gammas[i].reshape(1, h)
        ti = betas[i].reshape(1, h)
        if i < nlayers - 1:
            dv_n = dinv1 if (i + 1) < half else dinv2
            mta, mtb = _tc_layer(agga, aggb, mta, mtb, dv_p, dv_n,
                                 bi, gi, ti, W_convs[i + 1])
        else:
            out = _tc_final(agga, aggb, mta, mtb, dv_p, bi, gi, ti,
                            w_out_p, b_out_p, c_out)
    return out


# speedup vs baseline: 2.4482x; 2.4482x over previous
import functools

import jax
import jax.numpy as jnp
import numpy as np
from jax import lax
from jax.experimental import pallas as pl
from jax.experimental.pallas import tpu as pltpu
from jax.experimental.pallas import tpu_sc as plsc

NC = 2
NS = 16
LN = 16
CH = 128

_BNR = float(1.0 / np.sqrt(1.0 + 1e-5))


@functools.lru_cache(maxsize=None)
def _make_agg_kernel(n, hh):
    stripe = n // NS
    arows = stripe + 1
    mesh = plsc.VectorSubcoreMesh(core_axis_name="c", subcore_axis_name="s",
                                  num_cores=NC, num_subcores=NS)

    @functools.partial(
        pl.kernel,
        out_type=jax.ShapeDtypeStruct((NC, NS, stripe, hh), jnp.float32),
        mesh=mesh,
        compiler_params=pltpu.CompilerParams(needs_layout_passes=False),
        scratch_types=[
            pltpu.VMEM((CH,), jnp.int32),
            pltpu.VMEM((CH,), jnp.int32),
            pltpu.VMEM((LN,), jnp.int32),
            pltpu.VMEM((CH, hh), jnp.float32),
            pltpu.VMEM((arows, hh), jnp.float32),
            pltpu.SemaphoreType.DMA,
        ],
    )
    def agg_kernel(mta_hbm, mtb_hbm, src_hbm, dst_hbm, lo_hbm, nch_hbm,
                   out_hbm, sidx_v, didx_v, bnd_v, rows_v, acc_v, sem):
        c = lax.axis_index("c")
        s = lax.axis_index("s")
        base = s * stripe
        iot = lax.iota(jnp.int32, LN)
        z16 = jnp.zeros((LN,), jnp.float32)

        def zrow(j, carry):
            for t in range(hh // LN):
                acc_v[j, pl.ds(t * LN, LN)] = z16
            return carry
        lax.fori_loop(0, arows, zrow, 0)

        pltpu.sync_copy(lo_hbm, bnd_v)
        lo8 = lax.reduce_max(jnp.where(iot == s, bnd_v[...], 0), (0,))
        pltpu.sync_copy(nch_hbm, bnd_v)
        nch = lax.reduce_max(jnp.where(iot == s, bnd_v[...], 0), (0,))

        def make_chunk_body(mt_hbm):
            def chunk_body(ch, carry):
                eb = pl.multiple_of(lo8 + ch * CH, 8)
                pltpu.sync_copy(src_hbm.at[pl.ds(eb, CH)], sidx_v)
                pltpu.sync_copy(dst_hbm.at[pl.ds(eb, CH)], didx_v)
                pltpu.async_copy(mt_hbm.at[sidx_v], rows_v, sem).wait()

                def grp_body(k, carry2):
                    dchunk = didx_v[pl.ds(k * LN, LN)]
                    dloc = dchunk - base
                    lrow = jnp.where((dloc >= 0) & (dloc < stripe),
                                     dloc, stripe)
                    for i in range(LN):
                        row = lax.reduce_max(
                            jnp.where(iot == i, lrow, 0), (0,))
                        e = k * LN + i
                        for t in range(hh // LN):
                            sl = pl.ds(t * LN, LN)
                            acc_v[row, sl] = acc_v[row, sl] + rows_v[e, sl]
                    return carry2

                lax.fori_loop(0, CH // LN, grp_body, 0)
                return carry
            return chunk_body

        @pl.when(c == 0)
        def _():
            lax.fori_loop(0, nch, make_chunk_body(mta_hbm), 0)

        @pl.when(c == 1)
        def _():
            lax.fori_loop(0, nch, make_chunk_body(mtb_hbm), 0)

        pltpu.sync_copy(acc_v.at[pl.ds(0, stripe)], out_hbm.at[c, s])

    return agg_kernel


def _sc_aggregate(mta, mtb, srcp, dstp, los8, nchs):
    n, hh = mta.shape
    return _make_agg_kernel(n, hh)(mta, mtb, srcp, dstp, los8, nchs)


_R = 2000


def _row_spec(r, width):
    return pl.BlockSpec((r, width), lambda i: (i, 0))


def _full_spec(shape):
    nd = len(shape)
    return pl.BlockSpec(shape, lambda i: (0,) * nd)


def _tc_dinv(deg):
    nc, n, _ = deg.shape
    r = min(_R, n)

    def body(deg_ref, o_ref):
        o_ref[...] = lax.rsqrt(deg_ref[...] + 1.0)

    return pl.pallas_call(
        body,
        grid=(nc, n // r),
        in_specs=[pl.BlockSpec((1, r, 1), lambda c, i: (c, i, 0))],
        out_specs=pl.BlockSpec((1, r, 1), lambda c, i: (c, i, 0)),
        out_shape=jax.ShapeDtypeStruct((nc, n, 1), jnp.float32),
    )(deg)


def _tc_input(x, w_in, b_in):
    n, d = x.shape
    h = w_in.shape[1]
    hh = h // 2
    r = min(_R, n)

    def body(x_ref, w_ref, b_ref, oa_ref, ob_ref):
        hm = jnp.dot(x_ref[...], w_ref[...],
                     preferred_element_type=jnp.float32) + b_ref[...]
        oa_ref[...] = hm[:, :hh]
        ob_ref[...] = hm[:, hh:]

    return pl.pallas_call(
        body,
        grid=(n // r,),
        in_specs=[_row_spec(r, d), _full_spec((d, h)), _full_spec((1, h))],
        out_specs=[_row_spec(r, hh), _row_spec(r, hh)],
        out_shape=[jax.ShapeDtypeStruct((n, hh), jnp.float32),
                   jax.ShapeDtypeStruct((n, hh), jnp.float32)],
    )(x, w_in, b_in)


def _tc_mt0(h0a, h0b, w, dinv):
    n, hh = h0a.shape
    h = w.shape[0]
    r = min(_R, n)

    def body(ha_ref, hb_ref, w_ref, dv_ref, oa_ref, ob_ref):
        m = (jnp.dot(ha_ref[...], w_ref[:hh, :],
                     preferred_element_type=jnp.float32) +
             jnp.dot(hb_ref[...], w_ref[hh:, :],
                     preferred_element_type=jnp.float32))
        mt = dv_ref[...] * m
        oa_ref[...] = mt[:, :hh]
        ob_ref[...] = mt[:, hh:]

    return pl.pallas_call(
        body,
        grid=(n // r,),
        in_specs=[_row_spec(r, hh), _row_spec(r, hh), _full_spec((h, h)),
                  _row_spec(r, 1)],
        out_specs=[_row_spec(r, hh), _row_spec(r, hh)],
        out_shape=[jax.ShapeDtypeStruct((n, hh), jnp.float32),
                   jax.ShapeDtypeStruct((n, hh), jnp.float32)],
    )(h0a, h0b, w, dinv)


def _tc_layer(agga, aggb, mta, mtb, dv_p, dv_n, b, g, bet, w):
    n, hh = agga.shape
    h = w.shape[0]
    r = min(_R, n)

    def body(aa_ref, ab_ref, ma_ref, mb_ref, dp_ref, dn_ref,
             b_ref, g_ref, t_ref, w_ref, oa_ref, ob_ref):
        sc = g_ref[...] * _BNR
        dp = dp_ref[...]
        ha = (dp * (aa_ref[...] + ma_ref[...]) + b_ref[:, :hh]) * sc[:, :hh] \
            + t_ref[:, :hh]
        hb = (dp * (ab_ref[...] + mb_ref[...]) + b_ref[:, hh:]) * sc[:, hh:] \
            + t_ref[:, hh:]
        ha = jnp.maximum(ha, 0.0)
        hb = jnp.maximum(hb, 0.0)
        m = (jnp.dot(ha, w_ref[:hh, :], preferred_element_type=jnp.float32) +
             jnp.dot(hb, w_ref[hh:, :], preferred_element_type=jnp.float32))
        mt = dn_ref[...] * m
        oa_ref[...] = mt[:, :hh]
        ob_ref[...] = mt[:, hh:]

    return pl.pallas_call(
        body,
        grid=(n // r,),
        in_specs=[_row_spec(r, hh), _row_spec(r, hh),
                  _row_spec(r, hh), _row_spec(r, hh),
                  _row_spec(r, 1), _row_spec(r, 1),
                  _full_spec((1, h)), _full_spec((1, h)), _full_spec((1, h)),
                  _full_spec((h, h))],
        out_specs=[_row_spec(r, hh), _row_spec(r, hh)],
        out_shape=[jax.ShapeDtypeStruct((n, hh), jnp.float32),
                   jax.ShapeDtypeStruct((n, hh), jnp.float32)],
    )(agga, aggb, mta, mtb, dv_p, dv_n, b, g, bet, w)


def _tc_final(agga, aggb, mta, mtb, dv_p, b, g, bet, w_out_p, b_out_p, c_out):
    n, hh = agga.shape
    h = w_out_p.shape[0]
    cp = w_out_p.shape[1]
    r = min(_R, n)

    def body(aa_ref, ab_ref, ma_ref, mb_ref, dp_ref,
             b_ref, g_ref, t_ref, wo_ref, bo_ref, o_ref):
        sc = g_ref[...] * _BNR
        dp = dp_ref[...]
        ha = (dp * (aa_ref[...] + ma_ref[...]) + b_ref[:, :hh]) * sc[:, :hh] \
            + t_ref[:, :hh]
        hb = (dp * (ab_ref[...] + mb_ref[...]) + b_ref[:, hh:]) * sc[:, hh:] \
            + t_ref[:, hh:]
        ha = jnp.maximum(ha, 0.0)
        hb = jnp.maximum(hb, 0.0)
        logits = (jnp.dot(ha, wo_ref[:hh, :],
                          preferred_element_type=jnp.float32) +
                  jnp.dot(hb, wo_ref[hh:, :],
                          preferred_element_type=jnp.float32) + bo_ref[...])
        mx = jnp.max(logits, axis=1, keepdims=True)
        sh = logits - mx
        lse = jnp.log(jnp.sum(jnp.exp(sh), axis=1, keepdims=True))
        o_ref[...] = (sh - lse)[:, :c_out]

    return pl.pallas_call(
        body,
        grid=(n // r,),
        in_specs=[_row_spec(r, hh), _row_spec(r, hh),
                  _row_spec(r, hh), _row_spec(r, hh),
                  _row_spec(r, 1),
                  _full_spec((1, h)), _full_spec((1, h)), _full_spec((1, h)),
                  _full_spec((h, cp)), _full_spec((1, cp))],
        out_specs=_row_spec(r, c_out),
        out_shape=jax.ShapeDtypeStruct((n, c_out), jnp.float32),
    )(agga, aggb, mta, mtb, dv_p, b, g, bet, w_out_p, b_out_p)


def _prep_edges(adj, n):
    dst_s, src_s = lax.sort_key_val(adj[1], adj[0])
    ss = jnp.searchsorted(dst_s, jnp.arange(n + 1, dtype=jnp.int32))
    deg = (ss[1:] - ss[:n]).astype(jnp.float32)
    stripe = n // NS
    starts = ss[jnp.arange(NS + 1) * stripe].astype(jnp.int32)
    los = starts[:NS]
    his = starts[1:]
    los8 = (los // 8) * 8
    nchs = (his - los8 + CH - 1) // CH
    src_p = jnp.concatenate([src_s, jnp.zeros((CH,), jnp.int32)])
    dst_p = jnp.concatenate([dst_s, jnp.full((CH,), jnp.int32(2 ** 30))])
    return src_p, dst_p, los8.astype(jnp.int32), nchs.astype(jnp.int32), deg


def kernel(x, sample1_adj, sample2_adj, W_in, b_in, W_convs, b_convs,
           gammas, betas, W_out, b_out):
    n, d = x.shape
    h = W_in.shape[1]
    nlayers = W_convs.shape[0]
    c_out = W_out.shape[1]

    e1 = _prep_edges(sample1_adj, n)
    e2 = _prep_edges(sample2_adj, n)
    deg = jnp.stack([e1[4], e2[4]])[:, :, None]
    dinv_all = _tc_dinv(deg)
    dinv1 = dinv_all[0]
    dinv2 = dinv_all[1]

    h0a, h0b = _tc_input(x, W_in, b_in.reshape(1, h))
    mta, mtb = _tc_mt0(h0a, h0b, W_convs[0], dinv1)

    cp = 128
    w_out_p = jnp.zeros((h, cp), jnp.float32).at[:, :c_out].set(W_out)
    b_out_p = jnp.full((1, cp), -1e30, jnp.float32).at[0, :c_out].set(b_out)

    half = nlayers // 2
    out = None
    for i in range(nlayers):
        srcp, dstp, los8, nchs, _ = e1 if i < half else e2
        dv_p = dinv1 if i < half else dinv2
        agg = _sc_aggregate(mta, mtb, srcp, dstp, los8, nchs)
        agga = agg[0].reshape(n, h // 2)
        aggb = agg[1].reshape(n, h // 2)
        bi = b_convs[i].reshape(1, h)
        gi = gammas[i].reshape(1, h)
        ti = betas[i].reshape(1, h)
        if i < nlayers - 1:
            dv_n = dinv1 if (i + 1) < half else dinv2
            mta, mtb = _tc_layer(agga, aggb, mta, mtb, dv_p, dv_n,
                                 bi, gi, ti, W_convs[i + 1])
        else:
            out = _tc_final(agga, aggb, mta, mtb, dv_p, bi, gi, ti,
                            w_out_p, b_out_p, c_out)
    return out
